# baseline (speedup 1.0000x reference)
import functools
import math

import jax
import jax.numpy as jnp
from jax import lax
from jax.experimental import pallas as pl
from jax.experimental.pallas import tpu as pltpu
from jax.experimental.pallas import tpu_sc as plsc

_CHUNKS = (128, 96, 96, 64, 64, 32, 16, 16)


def _make_gather(B, S, V, D, scale):
    info = plsc.get_sparse_core_info()
    NC, NS, L = info.num_cores, info.num_subcores, info.num_lanes
    NW = NC * NS
    N = B * S
    assert N % (8 * NW) == 0 and D % L == 0
    b_per_w = N // NW
    assert S % b_per_w == 0
    sizes = list(_CHUNKS)
    offs = [sum(sizes[:i]) for i in range(len(sizes))]
    nchunk = len(sizes)
    assert sum(sizes) == b_per_w and all(s % 8 == 0 for s in sizes)

    mesh = plsc.VectorSubcoreMesh(core_axis_name="c", subcore_axis_name="s")

    @functools.partial(
        pl.kernel,
        mesh=mesh,
        out_type=jax.ShapeDtypeStruct((N, D), jnp.float32),
        scratch_types=[
            pltpu.VMEM((b_per_w,), jnp.int32),
            pltpu.VMEM((b_per_w, D), jnp.float32),
        ]
        + [pltpu.SemaphoreType.DMA] * (nchunk + 3),
    )
    def emb_gather(idx_hbm, table_hbm, out_hbm, idx_v, rows_v, *sems):
        gsems, wsem = sems[:nchunk], sems[nchunk]
        isem0, isem1 = sems[nchunk + 1], sems[nchunk + 2]
        wid = lax.axis_index("s") * NC + lax.axis_index("c")
        base = wid * b_per_w
        row = base // S
        col = base % S
        i0 = pltpu.async_copy(
            idx_hbm.at[row, pl.ds(col, sizes[0])], idx_v.at[pl.ds(0, sizes[0])], isem0
        )
        i1 = pltpu.async_copy(
            idx_hbm.at[row, pl.ds(col + sizes[0], b_per_w - sizes[0])],
            idx_v.at[pl.ds(sizes[0], b_per_w - sizes[0])],
            isem1,
        )
        gathers = []
        for c in range(nchunk):
            if c == 0:
                i0.wait()
            elif c == 1:
                i1.wait()
            gathers.append(
                pltpu.async_copy(
                    table_hbm.at[idx_v.at[pl.ds(offs[c], sizes[c])]],
                    rows_v.at[pl.ds(offs[c], sizes[c])],
                    gsems[c],
                )
            )
        writes = []
        for c in range(nchunk):
            gathers[c].wait()

            def body(i, carry, _c=c):
                for j in range(D // L):
                    sl = (offs[_c] + i, pl.ds(j * L, L))
                    rows_v[sl] = rows_v[sl] * scale
                return carry

            lax.fori_loop(0, sizes[c], body, 0)
            writes.append(
                pltpu.async_copy(
                    rows_v.at[pl.ds(offs[c], sizes[c])],
                    out_hbm.at[pl.ds(base + offs[c], sizes[c])],
                    wsem,
                )
            )
        for w in writes:
            w.wait()

    return emb_gather


def kernel(token_ids, emb_table):
    B, S = token_ids.shape
    V, D = emb_table.shape
    scale = math.sqrt(float(D))
    if token_ids.dtype != jnp.int32:
        token_ids = token_ids.astype(jnp.int32)
    out = _make_gather(B, S, V, D, scale)(token_ids, emb_table)
    return out.reshape(B, S, D)


# speedup vs baseline: 1.0246x; 1.0246x over previous
import functools
import math

import jax
import jax.numpy as jnp
from jax import lax
from jax.experimental import pallas as pl
from jax.experimental.pallas import tpu as pltpu
from jax.experimental.pallas import tpu_sc as plsc

_CHUNKS = (64, 64, 64, 64, 64, 64, 64, 64)


def _make_gather(B, S, V, D, scale):
    info = plsc.get_sparse_core_info()
    NC, NS, L = info.num_cores, info.num_subcores, info.num_lanes
    NW = NC * NS
    N = B * S
    assert N % (8 * NW) == 0 and D % L == 0
    b_per_w = N // NW
    assert S % b_per_w == 0
    sizes = list(_CHUNKS)
    offs = [sum(sizes[:i]) for i in range(len(sizes))]
    nchunk = len(sizes)
    assert sum(sizes) == b_per_w and all(s % 8 == 0 for s in sizes)

    mesh = plsc.VectorSubcoreMesh(core_axis_name="c", subcore_axis_name="s")

    @functools.partial(
        pl.kernel,
        mesh=mesh,
        out_type=jax.ShapeDtypeStruct((N, D), jnp.float32),
        scratch_types=[
            pltpu.VMEM((b_per_w,), jnp.int32),
            pltpu.VMEM((b_per_w, D), jnp.float32),
        ]
        + [pltpu.SemaphoreType.DMA] * (nchunk + 1),
    )
    def emb_gather(idx_hbm, table_hbm, out_hbm, idx_v, rows_v, *sems):
        gsems, wsem = sems[:nchunk], sems[nchunk]
        wid = lax.axis_index("s") * NC + lax.axis_index("c")
        base = wid * b_per_w
        row = base // S
        col = base % S
        pltpu.sync_copy(idx_hbm.at[row, pl.ds(col, b_per_w)], idx_v)
        gathers = [
            pltpu.async_copy(
                table_hbm.at[idx_v.at[pl.ds(offs[c], sizes[c])]],
                rows_v.at[pl.ds(offs[c], sizes[c])],
                gsems[c],
            )
            for c in range(nchunk)
        ]
        writes = []
        for c in range(nchunk):
            gathers[c].wait()

            def body(i, carry, _c=c):
                for j in range(D // L):
                    sl = (offs[_c] + i, pl.ds(j * L, L))
                    rows_v[sl] = rows_v[sl] * scale
                return carry

            lax.fori_loop(0, sizes[c], body, 0)
            writes.append(
                pltpu.async_copy(
                    rows_v.at[pl.ds(offs[c], sizes[c])],
                    out_hbm.at[pl.ds(base + offs[c], sizes[c])],
                    wsem,
                )
            )
        for w in writes:
            w.wait()

    return emb_gather


def kernel(token_ids, emb_table):
    B, S = token_ids.shape
    V, D = emb_table.shape
    scale = math.sqrt(float(D))
    if token_ids.dtype != jnp.int32:
        token_ids = token_ids.astype(jnp.int32)
    out = _make_gather(B, S, V, D, scale)(token_ids, emb_table)
    return out.reshape(B, S, D)
